# baseline (speedup 1.0000x reference)
import functools

import jax
import jax.numpy as jnp
from jax.experimental import pallas as pl
from jax.experimental.pallas import tpu as pltpu

N = 4096
D = 128
H = 256

_HI = jax.lax.Precision.HIGHEST

_A_BLK = 512
_B_I = 256
_B_K = 512


def _pass_a_kernel(x_ref, w1_ref, b1_ref, adj_ref, w2_ref, z_ref, s_ref):
    i = pl.program_id(0)

    @pl.when(i == 0)
    def _():
        s_ref[...] = jnp.dot(x_ref[...], w1_ref[...], precision=_HI)

    h = jnp.dot(adj_ref[...], s_ref[...], precision=_HI) + b1_ref[...]
    h = jnp.where(h >= 0, h, 0.01 * h)
    z_ref[...] = jnp.dot(h, w2_ref[...], precision=_HI)


def _pass_b_kernel(adj_ref, z_ref, b2_ref, mask_ref, out_ref, acc_ref):
    k = pl.program_id(1)
    nk = pl.num_programs(1)

    part = jnp.dot(adj_ref[...], z_ref[...], precision=_HI)

    @pl.when(k == 0)
    def _():
        acc_ref[...] = part

    @pl.when(k > 0)
    def _():
        acc_ref[...] += part

    @pl.when(k == nk - 1)
    def _():
        acc_ref[...] += b2_ref[...]
        m = jnp.max(acc_ref[...], axis=1, keepdims=True)
        acc_ref[...] = jnp.exp(acc_ref[...] - m)
        denom = jnp.sum(acc_ref[...], axis=1, keepdims=True)
        out_ref[...] = acc_ref[...] / denom * mask_ref[...]


@functools.partial(jax.jit, static_argnames=())
def _run(x, adj, edge_mask, W1, b1, W2, b2):
    b1r = b1.reshape(1, H)
    b2r = b2.reshape(1, N)

    z = pl.pallas_call(
        _pass_a_kernel,
        grid=(N // _A_BLK,),
        in_specs=[
            pl.BlockSpec((N, D), lambda i: (0, 0)),
            pl.BlockSpec((D, H), lambda i: (0, 0)),
            pl.BlockSpec((1, H), lambda i: (0, 0)),
            pl.BlockSpec((_A_BLK, N), lambda i: (i, 0)),
            pl.BlockSpec((H, N), lambda i: (0, 0)),
        ],
        out_specs=pl.BlockSpec((_A_BLK, N), lambda i: (i, 0)),
        out_shape=jax.ShapeDtypeStruct((N, N), jnp.float32),
        scratch_shapes=[pltpu.VMEM((N, H), jnp.float32)],
    )(x, W1, b1r, adj, W2)

    out = pl.pallas_call(
        _pass_b_kernel,
        grid=(N // _B_I, N // _B_K),
        in_specs=[
            pl.BlockSpec((_B_I, _B_K), lambda i, k: (i, k)),
            pl.BlockSpec((_B_K, N), lambda i, k: (k, 0)),
            pl.BlockSpec((1, N), lambda i, k: (0, 0)),
            pl.BlockSpec((_B_I, N), lambda i, k: (i, 0)),
        ],
        out_specs=pl.BlockSpec((_B_I, N), lambda i, k: (i, 0)),
        out_shape=jax.ShapeDtypeStruct((N, N), jnp.float32),
        scratch_shapes=[pltpu.VMEM((_B_I, N), jnp.float32)],
    )(adj, z, b2r, edge_mask)

    return out


def kernel(x, adj, edge_mask, W1, b1, W2, b2, dropout):
    return _run(x, adj, edge_mask, W1, b1, W2, b2)
